# Initial kernel scaffold; baseline (speedup 1.0000x reference)
import jax
import jax.numpy as jnp
from jax.experimental import pallas as pl


def kernel(x, tok_table, pos_table):
    raise NotImplementedError("write your pallas kernel here")


# speedup vs baseline: 1.0247x; 1.0247x over previous
import functools

import jax
import jax.numpy as jnp
from jax import lax
from jax.experimental import pallas as pl
from jax.experimental.pallas import tpu as pltpu
from jax.experimental.pallas import tpu_sc as plsc

NC = 2
NS = 16
NW = NC * NS
LANES = 16

CHUNK = 128


def _embed_kernel(n_flat, maxlen, d, x_hbm, tok_hbm, pos2_hbm, out_hbm,
                  pos_v, idx_v, rows_v, sem_g, sem_o):
    wid = lax.axis_index("s") * NC + lax.axis_index("c")
    per_w = n_flat // NW
    n_chunks = per_w // CHUNK
    base = wid * per_w

    pltpu.sync_copy(pos2_hbm, pos_v)

    d_half = d // LANES

    @pl.loop(0, n_chunks)
    def _chunk(c):
        start = base + c * CHUNK
        start_pos = lax.rem(c * CHUNK, maxlen)

        pltpu.sync_copy(x_hbm.at[pl.ds(start, CHUNK)], idx_v)
        pltpu.async_copy(tok_hbm.at[idx_v], rows_v, sem_g).wait()

        @pl.loop(0, CHUNK)
        def _row(r):
            p = start_pos + r
            for h in range(d_half):
                plsc.addupdate(rows_v.at[r, pl.ds(h * LANES, LANES)],
                               pos_v[p, pl.ds(h * LANES, LANES)])

        pltpu.async_copy(rows_v, out_hbm.at[pl.ds(start, CHUNK)], sem_o).wait()


def kernel(x, tok_table, pos_table):
    batch, maxlen = x.shape
    vocab, d = tok_table.shape
    n_flat = batch * maxlen

    x_flat = x.reshape(n_flat).astype(jnp.int32)
    pos2 = jnp.concatenate([pos_table, pos_table], axis=0)

    mesh = plsc.VectorSubcoreMesh(core_axis_name="c", subcore_axis_name="s")
    run = pl.kernel(
        functools.partial(_embed_kernel, n_flat, maxlen, d),
        out_type=jax.ShapeDtypeStruct((n_flat, d), jnp.float32),
        mesh=mesh,
        scratch_types=[
            pltpu.VMEM((2 * maxlen, d), jnp.float32),
            pltpu.VMEM((CHUNK,), jnp.int32),
            pltpu.VMEM((CHUNK, d), jnp.float32),
            pltpu.SemaphoreType.DMA,
            pltpu.SemaphoreType.DMA,
        ],
        compiler_params=pltpu.CompilerParams(use_tc_tiling_on_sc=False),
    )
    out = run(x_flat, tok_table, pos2)
    return out.reshape(batch, maxlen, d)


# speedup vs baseline: 1.3135x; 1.2819x over previous
import functools

import jax
import jax.numpy as jnp
from jax import lax
from jax.experimental import pallas as pl
from jax.experimental.pallas import tpu as pltpu
from jax.experimental.pallas import tpu_sc as plsc

NC = 2
NS = 16
NW = NC * NS
LANES = 16

CHUNK = 128
NBUF = 4
DIST = 2


def _embed_kernel(n_chunks, maxlen, d, x_hbm, tok_hbm, pos2_hbm, out_hbm,
                  idx_v, pos_v, rows_v, sems_g, sems_o):
    wid = lax.axis_index("s") * NC + lax.axis_index("c")
    base = wid * n_chunks
    d_half = d // LANES

    pltpu.sync_copy(x_hbm.at[wid], idx_v)
    pltpu.sync_copy(pos2_hbm, pos_v)

    def fire_gather(c, b):
        pltpu.async_copy(tok_hbm.at[idx_v.at[c]], rows_v.at[b], sems_g[b])

    def wait_gather(b):
        pltpu.make_async_copy(tok_hbm.at[idx_v.at[0]], rows_v.at[b],
                              sems_g[b]).wait()

    def fire_out(c, b):
        pltpu.async_copy(rows_v.at[b], out_hbm.at[pl.ds((base + c) * CHUNK,
                                                        CHUNK)], sems_o[b])

    def wait_out(b):
        pltpu.make_async_copy(rows_v.at[b], out_hbm.at[pl.ds(0, CHUNK)],
                              sems_o[b]).wait()

    for c in range(DIST):
        fire_gather(c, c)

    @pl.loop(0, n_chunks, step=NBUF)
    def _group(c0):
        for b in range(NBUF):
            c = c0 + b
            wait_gather(b)

            start_pos = lax.rem(c * CHUNK, maxlen)

            @pl.loop(0, CHUNK, unroll=8)
            def _row(r):
                p = start_pos + r
                for h in range(d_half):
                    plsc.addupdate(rows_v.at[b, r, pl.ds(h * LANES, LANES)],
                                   pos_v[p, pl.ds(h * LANES, LANES)])

            fire_out(c, b)

            bg = (b + DIST) % NBUF

            @pl.when(c + DIST < n_chunks)
            def _():
                @pl.when(c >= NBUF - DIST)
                def _():
                    wait_out(bg)
                fire_gather(c + DIST, bg)

    for b in range(NBUF):
        wait_out(b)


def kernel(x, tok_table, pos_table):
    batch, maxlen = x.shape
    vocab, d = tok_table.shape
    n_flat = batch * maxlen
    per_w = n_flat // NW
    n_chunks = per_w // CHUNK

    x_split = x.reshape(NW, n_chunks, CHUNK).astype(jnp.int32)
    pos2 = jnp.concatenate([pos_table, pos_table], axis=0)

    mesh = plsc.VectorSubcoreMesh(core_axis_name="c", subcore_axis_name="s")
    run = pl.kernel(
        functools.partial(_embed_kernel, n_chunks, maxlen, d),
        out_type=jax.ShapeDtypeStruct((n_flat, d), jnp.float32),
        mesh=mesh,
        scratch_types=[
            pltpu.VMEM((n_chunks, CHUNK), jnp.int32),
            pltpu.VMEM((2 * maxlen, d), jnp.float32),
            pltpu.VMEM((NBUF, CHUNK, d), jnp.float32),
            [pltpu.SemaphoreType.DMA] * NBUF,
            [pltpu.SemaphoreType.DMA] * NBUF,
        ],
        compiler_params=pltpu.CompilerParams(use_tc_tiling_on_sc=False),
    )
    out = run(x_split, tok_table, pos2)
    return out.reshape(batch, maxlen, d)
